# baseline (speedup 1.0000x reference)
import jax
import jax.numpy as jnp
from jax import lax
from jax.experimental import pallas as pl

OMEGA_K = 256
NBITS = 31


def _thr_body(x_ref, o_ref):
    xb = x_ref[...]
    bits = lax.bitcast_convert_type(jnp.abs(xb), jnp.int32)
    ones = jnp.ones((8, xb.shape[0]), jnp.bfloat16)
    one_b = jnp.bfloat16(1.0)
    zero_b = jnp.bfloat16(0.0)

    def count(maskb):
        mf = jnp.where(maskb, one_b, zero_b)
        return jnp.dot(ones, mf, preferred_element_type=jnp.float32)[0]

    hi = (bits >> 16).astype(jnp.int16)
    lo = ((bits & 0xFFFF) ^ 0x8000).astype(jnp.int16)

    def step_hi(i, t):
        cand = t | (1 << (14 - i))
        cnt = count(hi >= cand.astype(jnp.int16)[None, :])
        return jnp.where(cnt >= float(OMEGA_K), cand, t)

    t_hi = lax.fori_loop(0, 15, step_hi, jnp.zeros((128,), jnp.int32))

    th16 = t_hi.astype(jnp.int16)[None, :]
    c_ge = count(hi >= th16)
    eq = hi == th16
    c_eq = count(eq)
    need = float(OMEGA_K) - (c_ge - c_eq)
    mlo = jnp.where(eq, lo, jnp.int16(-32768))

    def step_lo(i, t):
        cand = t | (1 << (15 - i))
        cnt = count(mlo >= (cand ^ 0x8000).astype(jnp.int16)[None, :])
        return jnp.where(cnt >= need, cand, t)

    t_lo = lax.fori_loop(0, 16, step_lo, jnp.zeros((128,), jnp.int32))

    thr = (t_hi << 16) | t_lo
    o_ref[...] = jnp.where(bits >= thr[None, :], xb, 0.0)


def kernel(x):
    b, w, d = x.shape
    return pl.pallas_call(
        _thr_body,
        grid=(b,),
        in_specs=[pl.BlockSpec((None, w, d), lambda i: (i, 0, 0))],
        out_specs=pl.BlockSpec((None, w, d), lambda i: (i, 0, 0)),
        out_shape=jax.ShapeDtypeStruct(x.shape, x.dtype),
    )(x)


# speedup vs baseline: 1.3029x; 1.3029x over previous
import jax
import jax.numpy as jnp
from jax import lax
from jax.experimental import pallas as pl

OMEGA_K = 256
NBITS = 31
BBLK = 2


def _thr_body(x_ref, o_ref):
    xb = x_ref[...]
    bits = lax.bitcast_convert_type(jnp.abs(xb), jnp.int32)
    w = xb.shape[1]
    ones = jnp.ones((BBLK, 8, w), jnp.bfloat16)
    dn = (((2,), (1,)), ((0,), (0,)))

    def step(i, t):
        cand = t | (1 << (30 - i))
        maskf = jnp.where(bits >= cand[:, None, :], 1.0, 0.0)
        cnt = lax.dot_general(ones, maskf.astype(jnp.bfloat16), dn,
                              preferred_element_type=jnp.float32)[:, 0, :]
        return jnp.where(cnt >= float(OMEGA_K), cand, t)

    t0 = jnp.zeros((BBLK, 128), jnp.int32)
    thr = lax.fori_loop(0, NBITS, step, t0)
    o_ref[...] = jnp.where(bits >= thr[:, None, :], xb, 0.0)


def kernel(x):
    b, w, d = x.shape
    return pl.pallas_call(
        _thr_body,
        grid=(b // BBLK,),
        in_specs=[pl.BlockSpec((BBLK, w, d), lambda i: (i, 0, 0))],
        out_specs=pl.BlockSpec((BBLK, w, d), lambda i: (i, 0, 0)),
        out_shape=jax.ShapeDtypeStruct(x.shape, x.dtype),
    )(x)


# speedup vs baseline: 1.4681x; 1.1268x over previous
import jax
import jax.numpy as jnp
from jax import lax
from jax.experimental import pallas as pl

OMEGA_K = 256
NBITS = 31
BBLK = 4


def _thr_body(x_ref, o_ref):
    xb = x_ref[...]
    bits = lax.bitcast_convert_type(jnp.abs(xb), jnp.int32)
    w = xb.shape[1]
    ones = jnp.ones((BBLK, 8, w), jnp.bfloat16)
    dn = (((2,), (1,)), ((0,), (0,)))

    def step(i, t):
        cand = t | (1 << (30 - i))
        maskf = jnp.where(bits >= cand[:, None, :], 1.0, 0.0)
        cnt = lax.dot_general(ones, maskf.astype(jnp.bfloat16), dn,
                              preferred_element_type=jnp.float32)[:, 0, :]
        return jnp.where(cnt >= float(OMEGA_K), cand, t)

    t0 = jnp.zeros((BBLK, 128), jnp.int32)
    thr = lax.fori_loop(0, NBITS, step, t0)
    o_ref[...] = jnp.where(bits >= thr[:, None, :], xb, 0.0)


def kernel(x):
    b, w, d = x.shape
    return pl.pallas_call(
        _thr_body,
        grid=(b // BBLK,),
        in_specs=[pl.BlockSpec((BBLK, w, d), lambda i: (i, 0, 0))],
        out_specs=pl.BlockSpec((BBLK, w, d), lambda i: (i, 0, 0)),
        out_shape=jax.ShapeDtypeStruct(x.shape, x.dtype),
    )(x)
